# baseline (speedup 1.0000x reference)
import functools

import jax
import jax.numpy as jnp
from jax import lax
from jax.experimental import pallas as pl
from jax.experimental.pallas import tpu as pltpu
from jax.experimental.pallas import tpu_sc as plsc

_B = 16384
_L = 50
_D = 64
_NC = 2
_NS = 16
_NW = _NC * _NS
_RPW = _B // _NW
_NBUF = 4


_LP = 128
_LG = 56


def _sc_gather_sums(table, xp):
  mesh = plsc.VectorSubcoreMesh(core_axis_name="c", subcore_axis_name="s")

  @functools.partial(
      pl.kernel,
      out_type=jax.ShapeDtypeStruct((_B, _D), jnp.float32),
      mesh=mesh,
      compiler_params=pltpu.CompilerParams(use_tc_tiling_on_sc=False),
      scratch_types=[
          pltpu.VMEM((_RPW, _L), jnp.int32),
          pltpu.VMEM((_RPW, _L), jnp.int32),
          pltpu.VMEM((_NBUF, _L, _D), jnp.float32),
          pltpu.VMEM((_RPW, _D), jnp.float32),
          pltpu.SemaphoreType.DMA,
          pltpu.SemaphoreType.DMA,
          pltpu.SemaphoreType.DMA,
          pltpu.SemaphoreType.DMA,
      ],
  )
  def sc_kernel(table_hbm, x_hbm, out_hbm, idx_v, idx2_v, bufs, sums_v,
                s0, s1, s2, s3):
    sems = (s0, s1, s2, s3)
    wid = lax.axis_index("c") * _NS + lax.axis_index("s")

    pltpu.sync_copy(x_hbm.at[pl.ds(wid * _RPW, _RPW), :], idx_v)

    def dbl(r, carry):
      for c in (0, 16, 32, _L - 16):
        idx2_v[r, pl.ds(c, 16)] = idx_v[r, pl.ds(c, 16)] * 2
      return carry
    lax.fori_loop(0, _RPW, dbl, 0)

    for b in range(_NBUF):
      pltpu.async_copy(
          table_hbm.at[idx2_v.at[b]], bufs.at[b], sems[b])

    def outer(i, carry):
      j0 = i * _NBUF
      for b in range(_NBUF):
        j = j0 + b
        pltpu.make_async_copy(
            table_hbm.at[idx2_v.at[j]], bufs.at[b], sems[b]).wait()
        buf = bufs.at[b]

        def body(k, acc):
          row = k * 5
          for u in range(5):
            acc = tuple(
                acc[c] + buf[row + u, pl.ds(c * 16, 16)]
                for c in range(_D // 16))
          return acc
        acc = lax.fori_loop(
            0, _L // 5, body,
            tuple(jnp.zeros((16,), jnp.float32) for _ in range(_D // 16)))
        for c in range(_D // 16):
          sums_v[j, pl.ds(c * 16, 16)] = acc[c]
        nxt = j + _NBUF

        @pl.when(nxt < _RPW)
        def _():
          pltpu.async_copy(table_hbm.at[idx2_v.at[nxt]], bufs.at[b], sems[b])
      return carry

    lax.fori_loop(0, _RPW // _NBUF, outer, 0)

    pltpu.sync_copy(sums_v, out_hbm.at[pl.ds(wid * _RPW, _RPW)])

  return sc_kernel(table, xp)


def _halfpad_body(t_ref, o_ref):
  t = t_ref[...]
  o_ref[...] = jnp.concatenate([t, t], axis=1)


def _tc_halfpad(table):
  v = table.shape[0]
  blk = 8000
  return pl.pallas_call(
      _halfpad_body,
      grid=(v // blk,),
      in_specs=[pl.BlockSpec((blk, _D), lambda i: (i, 0))],
      out_specs=pl.BlockSpec((blk, 2 * _D), lambda i: (i, 0)),
      out_shape=jax.ShapeDtypeStruct((v, 2 * _D), jnp.float32),
  )(table)


_BLK = 2048


def _mlp_body(s_ref, x_ref, t0_ref, w0_ref, b0_ref, wh_ref, bh_ref,
              wout_ref, bout_ref, o_ref):
  cnt0 = jnp.sum((x_ref[...] == 0).astype(jnp.float32), axis=1, keepdims=True)
  emb = (s_ref[...] - cnt0 * t0_ref[...]) * (1.0 / _L)
  h = jnp.dot(emb, w0_ref[...], preferred_element_type=jnp.float32) + b0_ref[...]
  for _ in range(2):
    h = jnp.dot(jnp.maximum(h, 0.0), wh_ref[...],
                preferred_element_type=jnp.float32) + bh_ref[...]
  h = jnp.maximum(h, 0.0)
  o_ref[...] = jnp.dot(h, wout_ref[...],
                       preferred_element_type=jnp.float32) + bout_ref[...]


def _tc_mlp(sums, x, t0, W0, b0, Wh, bh, Wout, bout):
  h = W0.shape[1]
  grid = (_B // _BLK,)
  return pl.pallas_call(
      _mlp_body,
      grid=grid,
      in_specs=[
          pl.BlockSpec((_BLK, _D), lambda i: (i, 0)),
          pl.BlockSpec((_BLK, _L), lambda i: (i, 0)),
          pl.BlockSpec((1, _D), lambda i: (0, 0)),
          pl.BlockSpec((_D, h), lambda i: (0, 0)),
          pl.BlockSpec((1, h), lambda i: (0, 0)),
          pl.BlockSpec((h, h), lambda i: (0, 0)),
          pl.BlockSpec((1, h), lambda i: (0, 0)),
          pl.BlockSpec((h, 1), lambda i: (0, 0)),
          pl.BlockSpec((1, 1), lambda i: (0, 0)),
      ],
      out_specs=pl.BlockSpec((_BLK, 1), lambda i: (i, 0)),
      out_shape=jax.ShapeDtypeStruct((_B, 1), jnp.float32),
  )(sums, x, t0, W0, b0, Wh, bh, Wout, bout)


def _kernel_impl(x, table, W0, b0, Wh, bh, Wout, bout):
  x = x.astype(jnp.int32)
  tp = _tc_halfpad(table).reshape(2 * table.shape[0], _D)
  sums = _sc_gather_sums(tp, x)
  return _tc_mlp(sums, x, table[0:1, :], W0, b0.reshape(1, -1), Wh,
                 bh.reshape(1, -1), Wout, bout.reshape(1, -1))


kernel = jax.jit(_kernel_impl)


# speedup vs baseline: 1.2069x; 1.2069x over previous
import functools

import jax
import jax.numpy as jnp
from jax import lax
from jax.experimental import pallas as pl
from jax.experimental.pallas import tpu as pltpu
from jax.experimental.pallas import tpu_sc as plsc

_B = 16384
_L = 50
_D = 64
_NC = 2
_NS = 16
_NW = _NC * _NS
_RPW = _B // _NW
_NBUF = 4


_LP = 128
_LG = 56


def _sc_gather_sums(table, xp):
  mesh = plsc.VectorSubcoreMesh(core_axis_name="c", subcore_axis_name="s")

  @functools.partial(
      pl.kernel,
      out_type=jax.ShapeDtypeStruct((_B, _D), jnp.float32),
      mesh=mesh,
      compiler_params=pltpu.CompilerParams(use_tc_tiling_on_sc=False),
      scratch_types=[
          pltpu.VMEM((_RPW, _L), jnp.int32),
          pltpu.VMEM((_RPW, _L), jnp.int32),
          pltpu.VMEM((_NBUF, _L, _D), jnp.float32),
          pltpu.VMEM((_RPW, _D), jnp.float32),
          pltpu.SemaphoreType.DMA,
          pltpu.SemaphoreType.DMA,
          pltpu.SemaphoreType.DMA,
          pltpu.SemaphoreType.DMA,
      ],
  )
  def sc_kernel(table_hbm, x_hbm, out_hbm, idx_v, idx2_v, bufs, sums_v,
                s0, s1, s2, s3):
    sems = (s0, s1, s2, s3)
    wid = lax.axis_index("c") * _NS + lax.axis_index("s")

    pltpu.sync_copy(x_hbm.at[pl.ds(wid * _RPW, _RPW), :], idx_v)

    def dbl(r, carry):
      for c in (0, 16, 32, _L - 16):
        idx2_v[r, pl.ds(c, 16)] = idx_v[r, pl.ds(c, 16)] * 2
      return carry
    lax.fori_loop(0, _RPW, dbl, 0)

    for b in range(_NBUF):
      pltpu.async_copy(
          table_hbm.at[idx2_v.at[b]], bufs.at[b], sems[b])

    def outer(i, carry):
      j0 = i * _NBUF
      for b in range(_NBUF):
        j = j0 + b
        pltpu.make_async_copy(
            table_hbm.at[idx2_v.at[j]], bufs.at[b], sems[b]).wait()
        buf = bufs.at[b]

        def body(k, acc):
          row = k * 5
          for u in range(5):
            acc = tuple(
                acc[c] + buf[row + u, pl.ds(c * 16, 16)]
                for c in range(_D // 16))
          return acc
        acc = lax.fori_loop(
            0, _L // 5, body,
            tuple(jnp.zeros((16,), jnp.float32) for _ in range(_D // 16)))
        for c in range(_D // 16):
          sums_v[j, pl.ds(c * 16, 16)] = acc[c]
        nxt = j + _NBUF

        @pl.when(nxt < _RPW)
        def _():
          pltpu.async_copy(table_hbm.at[idx2_v.at[nxt]], bufs.at[b], sems[b])
      return carry

    lax.fori_loop(0, _RPW // _NBUF, outer, 0)

    pltpu.sync_copy(sums_v, out_hbm.at[pl.ds(wid * _RPW, _RPW)])

  return sc_kernel(table, xp)


_BLK = 2048


def _mlp_body(s_ref, x_ref, t0_ref, w0_ref, b0_ref, wh_ref, bh_ref,
              wout_ref, bout_ref, o_ref):
  cnt0 = jnp.sum((x_ref[...] == 0).astype(jnp.float32), axis=1, keepdims=True)
  emb = (s_ref[...] - cnt0 * t0_ref[...]) * (1.0 / _L)
  h = jnp.dot(emb, w0_ref[...], preferred_element_type=jnp.float32) + b0_ref[...]
  for _ in range(2):
    h = jnp.dot(jnp.maximum(h, 0.0), wh_ref[...],
                preferred_element_type=jnp.float32) + bh_ref[...]
  h = jnp.maximum(h, 0.0)
  o_ref[...] = jnp.dot(h, wout_ref[...],
                       preferred_element_type=jnp.float32) + bout_ref[...]


def _tc_mlp(sums, x, t0, W0, b0, Wh, bh, Wout, bout):
  h = W0.shape[1]
  grid = (_B // _BLK,)
  return pl.pallas_call(
      _mlp_body,
      grid=grid,
      in_specs=[
          pl.BlockSpec((_BLK, _D), lambda i: (i, 0)),
          pl.BlockSpec((_BLK, _L), lambda i: (i, 0)),
          pl.BlockSpec((1, _D), lambda i: (0, 0)),
          pl.BlockSpec((_D, h), lambda i: (0, 0)),
          pl.BlockSpec((1, h), lambda i: (0, 0)),
          pl.BlockSpec((h, h), lambda i: (0, 0)),
          pl.BlockSpec((1, h), lambda i: (0, 0)),
          pl.BlockSpec((h, 1), lambda i: (0, 0)),
          pl.BlockSpec((1, 1), lambda i: (0, 0)),
      ],
      out_specs=pl.BlockSpec((_BLK, 1), lambda i: (i, 0)),
      out_shape=jax.ShapeDtypeStruct((_B, 1), jnp.float32),
  )(sums, x, t0, W0, b0, Wh, bh, Wout, bout)


def _kernel_impl(x, table, W0, b0, Wh, bh, Wout, bout):
  x = x.astype(jnp.int32)
  tp = jnp.pad(table, ((0, 0), (0, _D))).reshape(2 * table.shape[0], _D)
  sums = _sc_gather_sums(tp, x)
  return _tc_mlp(sums, x, table[0:1, :], W0, b0.reshape(1, -1), Wh,
                 bh.reshape(1, -1), Wout, bout.reshape(1, -1))


kernel = jax.jit(_kernel_impl)


# speedup vs baseline: 1.2895x; 1.0684x over previous
import functools

import jax
import jax.numpy as jnp
from jax import lax
from jax.experimental import pallas as pl
from jax.experimental.pallas import tpu as pltpu
from jax.experimental.pallas import tpu_sc as plsc

_B = 16384
_L = 50
_D = 64
_NC = 2
_NS = 16
_NW = _NC * _NS
_RPW = _B // _NW
_NBUF = 8


_LP = 128
_LG = 56


def _sc_gather_sums(table, xp):
  mesh = plsc.VectorSubcoreMesh(core_axis_name="c", subcore_axis_name="s")

  @functools.partial(
      pl.kernel,
      out_type=jax.ShapeDtypeStruct((_B, _D), jnp.float32),
      mesh=mesh,
      compiler_params=pltpu.CompilerParams(use_tc_tiling_on_sc=False),
      scratch_types=[
          pltpu.VMEM((_RPW, _L), jnp.int32),
          pltpu.VMEM((_RPW, _L), jnp.int32),
          pltpu.VMEM((_NBUF, _L, _D), jnp.float32),
          pltpu.VMEM((_RPW, _D), jnp.float32),
          pltpu.SemaphoreType.DMA,
          pltpu.SemaphoreType.DMA,
          pltpu.SemaphoreType.DMA,
          pltpu.SemaphoreType.DMA,
          pltpu.SemaphoreType.DMA,
          pltpu.SemaphoreType.DMA,
          pltpu.SemaphoreType.DMA,
          pltpu.SemaphoreType.DMA,
      ],
  )
  def sc_kernel(table_hbm, x_hbm, out_hbm, idx_v, idx2_v, bufs, sums_v,
                s0, s1, s2, s3, s4, s5, s6, s7):
    sems = (s0, s1, s2, s3, s4, s5, s6, s7)
    wid = lax.axis_index("c") * _NS + lax.axis_index("s")

    pltpu.sync_copy(x_hbm.at[pl.ds(wid * _RPW, _RPW), :], idx_v)

    def dbl(r, carry):
      for c in (0, 16, 32, _L - 16):
        idx2_v[r, pl.ds(c, 16)] = idx_v[r, pl.ds(c, 16)] * 2
      return carry
    lax.fori_loop(0, _RPW, dbl, 0)

    for b in range(_NBUF):
      pltpu.async_copy(
          table_hbm.at[idx2_v.at[b]], bufs.at[b], sems[b])

    def outer(i, carry):
      j0 = i * _NBUF
      for b in range(_NBUF):
        j = j0 + b
        pltpu.make_async_copy(
            table_hbm.at[idx2_v.at[j]], bufs.at[b], sems[b]).wait()
        buf = bufs.at[b]

        def body(k, acc):
          row = k * 5
          for u in range(5):
            acc = tuple(
                acc[c] + buf[row + u, pl.ds(c * 16, 16)]
                for c in range(_D // 16))
          return acc
        acc = lax.fori_loop(
            0, _L // 5, body,
            tuple(jnp.zeros((16,), jnp.float32) for _ in range(_D // 16)))
        for c in range(_D // 16):
          sums_v[j, pl.ds(c * 16, 16)] = acc[c]
        nxt = j + _NBUF

        @pl.when(nxt < _RPW)
        def _():
          pltpu.async_copy(table_hbm.at[idx2_v.at[nxt]], bufs.at[b], sems[b])
      return carry

    lax.fori_loop(0, _RPW // _NBUF, outer, 0)

    pltpu.sync_copy(sums_v, out_hbm.at[pl.ds(wid * _RPW, _RPW)])

  return sc_kernel(table, xp)


_BLK = 2048


def _mlp_body(s_ref, x_ref, t0_ref, w0_ref, b0_ref, wh_ref, bh_ref,
              wout_ref, bout_ref, o_ref):
  cnt0 = jnp.sum((x_ref[...] == 0).astype(jnp.float32), axis=1, keepdims=True)
  emb = (s_ref[...] - cnt0 * t0_ref[...]) * (1.0 / _L)
  h = jnp.dot(emb, w0_ref[...], preferred_element_type=jnp.float32) + b0_ref[...]
  for _ in range(2):
    h = jnp.dot(jnp.maximum(h, 0.0), wh_ref[...],
                preferred_element_type=jnp.float32) + bh_ref[...]
  h = jnp.maximum(h, 0.0)
  o_ref[...] = jnp.dot(h, wout_ref[...],
                       preferred_element_type=jnp.float32) + bout_ref[...]


def _tc_mlp(sums, x, t0, W0, b0, Wh, bh, Wout, bout):
  h = W0.shape[1]
  grid = (_B // _BLK,)
  return pl.pallas_call(
      _mlp_body,
      grid=grid,
      in_specs=[
          pl.BlockSpec((_BLK, _D), lambda i: (i, 0)),
          pl.BlockSpec((_BLK, _L), lambda i: (i, 0)),
          pl.BlockSpec((1, _D), lambda i: (0, 0)),
          pl.BlockSpec((_D, h), lambda i: (0, 0)),
          pl.BlockSpec((1, h), lambda i: (0, 0)),
          pl.BlockSpec((h, h), lambda i: (0, 0)),
          pl.BlockSpec((1, h), lambda i: (0, 0)),
          pl.BlockSpec((h, 1), lambda i: (0, 0)),
          pl.BlockSpec((1, 1), lambda i: (0, 0)),
      ],
      out_specs=pl.BlockSpec((_BLK, 1), lambda i: (i, 0)),
      out_shape=jax.ShapeDtypeStruct((_B, 1), jnp.float32),
  )(sums, x, t0, W0, b0, Wh, bh, Wout, bout)


def _kernel_impl(x, table, W0, b0, Wh, bh, Wout, bout):
  x = x.astype(jnp.int32)
  tp = jnp.pad(table, ((0, 0), (0, _D))).reshape(2 * table.shape[0], _D)
  sums = _sc_gather_sums(tp, x)
  return _tc_mlp(sums, x, table[0:1, :], W0, b0.reshape(1, -1), Wh,
                 bh.reshape(1, -1), Wout, bout.reshape(1, -1))


kernel = jax.jit(_kernel_impl)


# speedup vs baseline: 1.2902x; 1.0005x over previous
import functools

import jax
import jax.numpy as jnp
from jax import lax
from jax.experimental import pallas as pl
from jax.experimental.pallas import tpu as pltpu
from jax.experimental.pallas import tpu_sc as plsc

_B = 16384
_L = 50
_D = 64
_NC = 2
_NS = 16
_NW = _NC * _NS
_RPW = _B // _NW
_NBUF = 8


_LP = 128
_LG = 56


def _sc_gather_sums(table, xp):
  mesh = plsc.VectorSubcoreMesh(core_axis_name="c", subcore_axis_name="s")

  @functools.partial(
      pl.kernel,
      out_type=jax.ShapeDtypeStruct((_B, _D), jnp.float32),
      mesh=mesh,
      compiler_params=pltpu.CompilerParams(use_tc_tiling_on_sc=False),
      scratch_types=[
          pltpu.VMEM((_RPW, _L), jnp.int32),
          pltpu.VMEM((_RPW, _L), jnp.int32),
          pltpu.VMEM((_NBUF, _L, _D), jnp.float32),
          pltpu.VMEM((_RPW, _D), jnp.float32),
      ] + [pltpu.SemaphoreType.DMA] * 8,
  )
  def sc_kernel(table_hbm, x_hbm, out_hbm, idx_v, idx2_v, bufs, sums_v,
                *sems):
    wid = lax.axis_index("c") * _NS + lax.axis_index("s")

    pltpu.sync_copy(x_hbm.at[pl.ds(wid * _RPW, _RPW), :], idx_v)

    def dbl(r, carry):
      for c in (0, 16, 32, _L - 16):
        idx2_v[r, pl.ds(c, 16)] = idx_v[r, pl.ds(c, 16)] * 2
      return carry
    lax.fori_loop(0, _RPW, dbl, 0)

    for b in range(_NBUF):
      pltpu.async_copy(
          table_hbm.at[idx2_v.at[b]], bufs.at[b], sems[b])

    def outer(i, carry):
      j0 = i * _NBUF
      for b in range(_NBUF):
        j = j0 + b
        pltpu.make_async_copy(
            table_hbm.at[idx2_v.at[j]], bufs.at[b], sems[b]).wait()
        buf = bufs.at[b]

        def body(k, acc):
          row = k * 5
          for u in range(5):
            acc = tuple(
                acc[c] + buf[row + u, pl.ds(c * 16, 16)]
                for c in range(_D // 16))
          return acc
        acc = lax.fori_loop(
            0, _L // 5, body,
            tuple(jnp.zeros((16,), jnp.float32) for _ in range(_D // 16)))
        for c in range(_D // 16):
          sums_v[j, pl.ds(c * 16, 16)] = acc[c]
        nxt = j + _NBUF

        @pl.when(nxt < _RPW)
        def _():
          pltpu.async_copy(table_hbm.at[idx2_v.at[nxt]], bufs.at[b], sems[b])
      return carry

    lax.fori_loop(0, _RPW // _NBUF, outer, 0)

    pltpu.sync_copy(sums_v, out_hbm.at[pl.ds(wid * _RPW, _RPW)])

  return sc_kernel(table, xp)


_BLK = 2048


def _mlp_body(s_ref, x_ref, t0_ref, w0_ref, b0_ref, wh_ref, bh_ref,
              wout_ref, bout_ref, o_ref):
  cnt0 = jnp.sum((x_ref[...] == 0).astype(jnp.float32), axis=1, keepdims=True)
  emb = (s_ref[...] - cnt0 * t0_ref[...]) * (1.0 / _L)
  h = jnp.dot(emb, w0_ref[...], preferred_element_type=jnp.float32) + b0_ref[...]
  for _ in range(2):
    h = jnp.dot(jnp.maximum(h, 0.0), wh_ref[...],
                preferred_element_type=jnp.float32) + bh_ref[...]
  h = jnp.maximum(h, 0.0)
  o_ref[...] = jnp.dot(h, wout_ref[...],
                       preferred_element_type=jnp.float32) + bout_ref[...]


def _tc_mlp(sums, x, t0, W0, b0, Wh, bh, Wout, bout):
  h = W0.shape[1]
  grid = (_B // _BLK,)
  return pl.pallas_call(
      _mlp_body,
      grid=grid,
      in_specs=[
          pl.BlockSpec((_BLK, _D), lambda i: (i, 0)),
          pl.BlockSpec((_BLK, _L), lambda i: (i, 0)),
          pl.BlockSpec((1, _D), lambda i: (0, 0)),
          pl.BlockSpec((_D, h), lambda i: (0, 0)),
          pl.BlockSpec((1, h), lambda i: (0, 0)),
          pl.BlockSpec((h, h), lambda i: (0, 0)),
          pl.BlockSpec((1, h), lambda i: (0, 0)),
          pl.BlockSpec((h, 1), lambda i: (0, 0)),
          pl.BlockSpec((1, 1), lambda i: (0, 0)),
      ],
      out_specs=pl.BlockSpec((_BLK, 1), lambda i: (i, 0)),
      out_shape=jax.ShapeDtypeStruct((_B, 1), jnp.float32),
  )(sums, x, t0, W0, b0, Wh, bh, Wout, bout)


def _kernel_impl(x, table, W0, b0, Wh, bh, Wout, bout):
  x = x.astype(jnp.int32)
  tp = jnp.pad(table, ((0, 0), (0, _D))).reshape(2 * table.shape[0], _D)
  sums = _sc_gather_sums(tp, x)
  return _tc_mlp(sums, x, table[0:1, :], W0, b0.reshape(1, -1), Wh,
                 bh.reshape(1, -1), Wout, bout.reshape(1, -1))


kernel = jax.jit(_kernel_impl)
